# Initial kernel scaffold; baseline (speedup 1.0000x reference)
import jax
import jax.numpy as jnp
from jax.experimental import pallas as pl


def kernel(x, W_router, W_gate, up, down):
    raise NotImplementedError("write your pallas kernel here")


# speedup vs baseline: 5.6654x; 5.6654x over previous
import jax
import jax.numpy as jnp
from jax.experimental import pallas as pl
from jax.experimental.pallas import tpu as pltpu

N, H, E, F = 512, 1024, 8, 256


def _moe_body(x_ref, wr_ref, wg_ref, up_ref, down_ref, out_ref, eid_ref,
              gate_ref):
    e = pl.program_id(0)
    x = x_ref[...]

    @pl.when(e == 0)
    def _init():
        logits = jax.lax.dot_general(
            x, wr_ref[...], (((1,), (1,)), ((), ())),
            precision=jax.lax.Precision.HIGHEST,
            preferred_element_type=jnp.float32)
        eid_ref[...] = jnp.argmax(logits, axis=1, keepdims=True).astype(
            jnp.int32)
        g = jax.lax.dot_general(
            x, wg_ref[...], (((1,), (1,)), ((), ())),
            precision=jax.lax.Precision.HIGHEST,
            preferred_element_type=jnp.float32)
        gate_ref[...] = jax.nn.sigmoid(g)
        out_ref[...] = jnp.zeros_like(out_ref)

    h = jax.lax.dot_general(
        x, up_ref[0], (((1,), (0,)), ((), ())),
        precision=jax.lax.Precision.HIGHEST,
        preferred_element_type=jnp.float32)
    act = h * jax.nn.sigmoid(h)
    act = jnp.where(eid_ref[...] == e, act, 0.0)
    contrib = jax.lax.dot_general(
        act, down_ref[0], (((1,), (0,)), ((), ())),
        precision=jax.lax.Precision.HIGHEST,
        preferred_element_type=jnp.float32)

    @pl.when(e < E - 1)
    def _acc():
        out_ref[...] += contrib

    @pl.when(e == E - 1)
    def _final():
        out_ref[...] = x + gate_ref[...] * (out_ref[...] + contrib)


def kernel(x, W_router, W_gate, up, down):
    out, eid = pl.pallas_call(
        _moe_body,
        grid=(E,),
        in_specs=[
            pl.BlockSpec((N, H), lambda e: (0, 0)),
            pl.BlockSpec((E, H), lambda e: (0, 0)),
            pl.BlockSpec((1, H), lambda e: (0, 0)),
            pl.BlockSpec((1, H, F), lambda e: (e, 0, 0)),
            pl.BlockSpec((1, F, H), lambda e: (e, 0, 0)),
        ],
        out_specs=[
            pl.BlockSpec((N, H), lambda e: (0, 0)),
            pl.BlockSpec((N, 1), lambda e: (0, 0)),
        ],
        out_shape=[
            jax.ShapeDtypeStruct((N, H), jnp.float32),
            jax.ShapeDtypeStruct((N, 1), jnp.int32),
        ],
        scratch_shapes=[pltpu.VMEM((N, 1), jnp.float32)],
    )(x, W_router, W_gate, up, down)
    return (out, eid.reshape(N))


# speedup vs baseline: 18.2196x; 3.2159x over previous
import jax
import jax.numpy as jnp
from jax.experimental import pallas as pl
from jax.experimental.pallas import tpu as pltpu

N, H, E, F = 512, 1024, 8, 256


def _moe_body(x_ref, wr_ref, wg_ref, up_ref, down_ref, out_ref, eid_ref,
              gate_ref):
    e = pl.program_id(0)
    x = x_ref[...]

    @pl.when(e == 0)
    def _init():
        logits = jax.lax.dot_general(
            x, wr_ref[...], (((1,), (1,)), ((), ())),
            preferred_element_type=jnp.float32)
        eid_ref[...] = jnp.argmax(logits, axis=1, keepdims=True).astype(
            jnp.int32)
        g = jax.lax.dot_general(
            x, wg_ref[...], (((1,), (1,)), ((), ())),
            precision=jax.lax.Precision.HIGHEST,
            preferred_element_type=jnp.float32)
        gate_ref[...] = jax.nn.sigmoid(g)
        out_ref[...] = jnp.zeros_like(out_ref)

    h = jax.lax.dot_general(
        x.astype(jnp.bfloat16), up_ref[0].astype(jnp.bfloat16),
        (((1,), (0,)), ((), ())),
        preferred_element_type=jnp.float32)
    act = h * jax.nn.sigmoid(h)
    act = jnp.where(eid_ref[...] == e, act, 0.0)
    contrib = jax.lax.dot_general(
        act.astype(jnp.bfloat16), down_ref[0].astype(jnp.bfloat16),
        (((1,), (0,)), ((), ())),
        preferred_element_type=jnp.float32)

    @pl.when(e < E - 1)
    def _acc():
        out_ref[...] += contrib

    @pl.when(e == E - 1)
    def _final():
        out_ref[...] = x + gate_ref[...] * (out_ref[...] + contrib)


def kernel(x, W_router, W_gate, up, down):
    out, eid = pl.pallas_call(
        _moe_body,
        grid=(E,),
        in_specs=[
            pl.BlockSpec((N, H), lambda e: (0, 0)),
            pl.BlockSpec((E, H), lambda e: (0, 0)),
            pl.BlockSpec((1, H), lambda e: (0, 0)),
            pl.BlockSpec((1, H, F), lambda e: (e, 0, 0)),
            pl.BlockSpec((1, F, H), lambda e: (e, 0, 0)),
        ],
        out_specs=[
            pl.BlockSpec((N, H), lambda e: (0, 0)),
            pl.BlockSpec((N, 1), lambda e: (0, 0)),
        ],
        out_shape=[
            jax.ShapeDtypeStruct((N, H), jnp.float32),
            jax.ShapeDtypeStruct((N, 1), jnp.int32),
        ],
        scratch_shapes=[pltpu.VMEM((N, 1), jnp.float32)],
    )(x, W_router, W_gate, up, down)
    return (out, eid.reshape(N))


# speedup vs baseline: 19.2245x; 1.0552x over previous
import jax
import jax.numpy as jnp
from jax.experimental import pallas as pl
from jax.experimental.pallas import tpu as pltpu

N, H, E, F = 512, 1024, 8, 256


def _moe_body(x_ref, wr_ref, wg_ref, up_ref, down_ref, out_ref, eid_ref,
              gate_ref, xb_ref):
    e = pl.program_id(0)

    @pl.when(e == 0)
    def _init():
        x = x_ref[...]
        xb_ref[...] = x.astype(jnp.bfloat16)
        logits = jax.lax.dot_general(
            x, wr_ref[...], (((1,), (1,)), ((), ())),
            preferred_element_type=jnp.float32)
        eid_ref[...] = jnp.argmax(logits, axis=1, keepdims=True).astype(
            jnp.int32)
        g = jax.lax.dot_general(
            x, wg_ref[...], (((1,), (1,)), ((), ())),
            precision=jax.lax.Precision.HIGHEST,
            preferred_element_type=jnp.float32)
        gate_ref[...] = jax.nn.sigmoid(g)
        out_ref[...] = jnp.zeros_like(out_ref)

    h = jax.lax.dot_general(
        xb_ref[...], up_ref[0].astype(jnp.bfloat16),
        (((1,), (0,)), ((), ())),
        preferred_element_type=jnp.float32)
    act = h * jax.nn.sigmoid(h)
    act = jnp.where(eid_ref[...] == e, act, 0.0)
    contrib = jax.lax.dot_general(
        act.astype(jnp.bfloat16), down_ref[0].astype(jnp.bfloat16),
        (((1,), (0,)), ((), ())),
        preferred_element_type=jnp.float32)

    @pl.when(e < E - 1)
    def _acc():
        out_ref[...] += contrib

    @pl.when(e == E - 1)
    def _final():
        out_ref[...] = x_ref[...] + gate_ref[...] * (out_ref[...] + contrib)


def kernel(x, W_router, W_gate, up, down):
    out, eid = pl.pallas_call(
        _moe_body,
        grid=(E,),
        in_specs=[
            pl.BlockSpec((N, H), lambda e: (0, 0)),
            pl.BlockSpec((E, H), lambda e: (0, 0)),
            pl.BlockSpec((1, H), lambda e: (0, 0)),
            pl.BlockSpec((1, H, F), lambda e: (e, 0, 0)),
            pl.BlockSpec((1, F, H), lambda e: (e, 0, 0)),
        ],
        out_specs=[
            pl.BlockSpec((N, H), lambda e: (0, 0)),
            pl.BlockSpec((N, 1), lambda e: (0, 0)),
        ],
        out_shape=[
            jax.ShapeDtypeStruct((N, H), jnp.float32),
            jax.ShapeDtypeStruct((N, 1), jnp.int32),
        ],
        scratch_shapes=[pltpu.VMEM((N, 1), jnp.float32),
                        pltpu.VMEM((N, H), jnp.bfloat16)],
    )(x, W_router, W_gate, up, down)
    return (out, eid.reshape(N))
